# baseline (speedup 1.0000x reference)
import jax
import jax.numpy as jnp
from jax.experimental import pallas as pl
from jax.experimental.pallas import tpu as pltpu

_LANE = 128


def _combine_kernel(w_ref, x_ref, o_ref):
    o_ref[...] = jnp.dot(
        w_ref[...],
        x_ref[...].astype(jnp.bfloat16),
        preferred_element_type=jnp.float32,
    )


def _combine_kernel_batched(w_ref, x_ref, o_ref):
    w = w_ref[...]
    for bb in range(x_ref.shape[0]):
        o_ref[bb] = jnp.dot(
            w,
            x_ref[bb].astype(jnp.bfloat16),
            preferred_element_type=jnp.float32,
        )


def _cost_estimate(b, c, new_c, t):
    return pl.CostEstimate(
        flops=2 * b * new_c * c * t,
        transcendentals=0,
        bytes_accessed=4 * (b * c * t + b * new_c * t) + 2 * new_c * c,
    )


def _combine_batched(weight16, x, tile_t, batch_block):
    new_c, c = weight16.shape
    b, _, t = x.shape
    grid_t = pl.cdiv(t, tile_t)
    grid_b = pl.cdiv(b, batch_block)
    return pl.pallas_call(
        _combine_kernel_batched,
        out_shape=jax.ShapeDtypeStruct((b, new_c, t), x.dtype),
        grid=(grid_b, grid_t),
        in_specs=[
            pl.BlockSpec((new_c, c), lambda i, j: (0, 0)),
            pl.BlockSpec((batch_block, c, tile_t), lambda i, j: (i, 0, j)),
        ],
        out_specs=pl.BlockSpec((batch_block, new_c, tile_t), lambda i, j: (i, 0, j)),
        compiler_params=pltpu.CompilerParams(
            dimension_semantics=("parallel", "parallel"),
        ),
        cost_estimate=_cost_estimate(b, c, new_c, t),
    )(weight16, x)


def _combine_folded(weight16, x, tile_l):
    new_c, c = weight16.shape
    b, _, t = x.shape
    lanes = b * t
    x2 = jnp.transpose(x, (1, 0, 2)).reshape(c, lanes)
    grid_l = pl.cdiv(lanes, tile_l)
    out2 = pl.pallas_call(
        _combine_kernel,
        out_shape=jax.ShapeDtypeStruct((new_c, lanes), x.dtype),
        grid=(grid_l,),
        in_specs=[
            pl.BlockSpec((new_c, c), lambda j: (0, 0)),
            pl.BlockSpec((c, tile_l), lambda j: (0, j)),
        ],
        out_specs=pl.BlockSpec((new_c, tile_l), lambda j: (0, j)),
        compiler_params=pltpu.CompilerParams(
            dimension_semantics=("parallel",),
        ),
        cost_estimate=_cost_estimate(b, c, new_c, t),
    )(weight16, x2)
    return jnp.transpose(out2.reshape(new_c, b, t), (1, 0, 2))


def kernel(weight, x):
    new_c, c = weight.shape
    b, c2, t = x.shape
    assert c == c2, "channel mismatch"
    weight16 = weight.astype(jnp.bfloat16)
    if t % _LANE == 0:
        tile_t = min(t, 2048)
        batch_block = 8 if b % 8 == 0 else (4 if b % 4 == 0 else 1)
        return _combine_batched(weight16, x, tile_t, batch_block)
    tile_l = min(b * t, 1024)
    return _combine_folded(weight16, x, tile_l)


# speedup vs baseline: 1.0045x; 1.0045x over previous
import jax
import jax.numpy as jnp
from jax.experimental import pallas as pl
from jax.experimental.pallas import tpu as pltpu

_LANE = 128


def _combine_kernel(w_ref, x_ref, o_ref):
    o_ref[...] = jnp.dot(
        w_ref[...],
        x_ref[...].astype(jnp.bfloat16),
        preferred_element_type=jnp.float32,
    )


def _combine_kernel_batched(w_ref, x_ref, o_ref):
    w = w_ref[...]
    for bb in range(x_ref.shape[0]):
        o_ref[bb] = jnp.dot(
            w,
            x_ref[bb].astype(jnp.bfloat16),
            preferred_element_type=jnp.float32,
        )


def _cost_estimate(b, c, new_c, t):
    return pl.CostEstimate(
        flops=2 * b * new_c * c * t,
        transcendentals=0,
        bytes_accessed=4 * (b * c * t + b * new_c * t) + 2 * new_c * c,
    )


def _combine_batched(weight16, x, tile_t, batch_block):
    new_c, c = weight16.shape
    b, _, t = x.shape
    grid_t = pl.cdiv(t, tile_t)
    grid_b = pl.cdiv(b, batch_block)
    return pl.pallas_call(
        _combine_kernel_batched,
        out_shape=jax.ShapeDtypeStruct((b, new_c, t), x.dtype),
        grid=(grid_b, grid_t),
        in_specs=[
            pl.BlockSpec((new_c, c), lambda i, j: (0, 0)),
            pl.BlockSpec((batch_block, c, tile_t), lambda i, j: (i, 0, j)),
        ],
        out_specs=pl.BlockSpec((batch_block, new_c, tile_t), lambda i, j: (i, 0, j)),
        compiler_params=pltpu.CompilerParams(
            dimension_semantics=("parallel", "parallel"),
        ),
        cost_estimate=_cost_estimate(b, c, new_c, t),
    )(weight16, x)


def _combine_folded(weight16, x, tile_l):
    new_c, c = weight16.shape
    b, _, t = x.shape
    lanes = b * t
    x2 = jnp.transpose(x, (1, 0, 2)).reshape(c, lanes)
    grid_l = pl.cdiv(lanes, tile_l)
    out2 = pl.pallas_call(
        _combine_kernel,
        out_shape=jax.ShapeDtypeStruct((new_c, lanes), x.dtype),
        grid=(grid_l,),
        in_specs=[
            pl.BlockSpec((new_c, c), lambda j: (0, 0)),
            pl.BlockSpec((c, tile_l), lambda j: (0, j)),
        ],
        out_specs=pl.BlockSpec((new_c, tile_l), lambda j: (0, j)),
        compiler_params=pltpu.CompilerParams(
            dimension_semantics=("parallel",),
        ),
        cost_estimate=_cost_estimate(b, c, new_c, t),
    )(weight16, x2)
    return jnp.transpose(out2.reshape(new_c, b, t), (1, 0, 2))


def kernel(weight, x):
    new_c, c = weight.shape
    b, c2, t = x.shape
    assert c == c2, "channel mismatch"
    weight16 = weight.astype(jnp.bfloat16)
    if t % _LANE == 0:
        tile_t = min(t, 2048)
        batch_block = 4 if b % 4 == 0 else 1
        return _combine_batched(weight16, x, tile_t, batch_block)
    tile_l = min(b * t, 1024)
    return _combine_folded(weight16, x, tile_l)


# speedup vs baseline: 1.0317x; 1.0271x over previous
import jax
import jax.numpy as jnp
from jax.experimental import pallas as pl
from jax.experimental.pallas import tpu as pltpu

_LANE = 128


def _combine_kernel(w_ref, x_ref, o_ref):
    o_ref[...] = jnp.dot(
        w_ref[...],
        x_ref[...].astype(jnp.bfloat16),
        preferred_element_type=jnp.float32,
    )


def _combine_kernel_batched(w_ref, x_ref, o_ref):
    w = w_ref[...].astype(jnp.bfloat16)
    for bb in range(x_ref.shape[0]):
        o_ref[bb] = jnp.dot(
            w,
            x_ref[bb].astype(jnp.bfloat16),
            preferred_element_type=jnp.float32,
        )


def _cost_estimate(b, c, new_c, t):
    return pl.CostEstimate(
        flops=2 * b * new_c * c * t,
        transcendentals=0,
        bytes_accessed=4 * (b * c * t + b * new_c * t) + 2 * new_c * c,
    )


def _combine_batched(weight16, x, tile_t, batch_block):
    new_c, c = weight16.shape
    b, _, t = x.shape
    grid_t = pl.cdiv(t, tile_t)
    grid_b = pl.cdiv(b, batch_block)
    return pl.pallas_call(
        _combine_kernel_batched,
        out_shape=jax.ShapeDtypeStruct((b, new_c, t), x.dtype),
        grid=(grid_b, grid_t),
        in_specs=[
            pl.BlockSpec((new_c, c), lambda i, j: (0, 0)),
            pl.BlockSpec((batch_block, c, tile_t), lambda i, j: (i, 0, j)),
        ],
        out_specs=pl.BlockSpec((batch_block, new_c, tile_t), lambda i, j: (i, 0, j)),
        compiler_params=pltpu.CompilerParams(
            dimension_semantics=("parallel", "parallel"),
        ),
        cost_estimate=_cost_estimate(b, c, new_c, t),
    )(weight16, x)


def _combine_folded(weight16, x, tile_l):
    new_c, c = weight16.shape
    b, _, t = x.shape
    lanes = b * t
    x2 = jnp.transpose(x, (1, 0, 2)).reshape(c, lanes)
    grid_l = pl.cdiv(lanes, tile_l)
    out2 = pl.pallas_call(
        _combine_kernel,
        out_shape=jax.ShapeDtypeStruct((new_c, lanes), x.dtype),
        grid=(grid_l,),
        in_specs=[
            pl.BlockSpec((new_c, c), lambda j: (0, 0)),
            pl.BlockSpec((c, tile_l), lambda j: (0, j)),
        ],
        out_specs=pl.BlockSpec((new_c, tile_l), lambda j: (0, j)),
        compiler_params=pltpu.CompilerParams(
            dimension_semantics=("parallel",),
        ),
        cost_estimate=_cost_estimate(b, c, new_c, t),
    )(weight16, x2)
    return jnp.transpose(out2.reshape(new_c, b, t), (1, 0, 2))


def kernel(weight, x):
    new_c, c = weight.shape
    b, c2, t = x.shape
    assert c == c2, "channel mismatch"
    if t % _LANE == 0:
        tile_t = min(t, 2048)
        batch_block = 4 if b % 4 == 0 else 1
        return _combine_batched(weight, x, tile_t, batch_block)
    tile_l = min(b * t, 1024)
    return _combine_folded(weight.astype(jnp.bfloat16), x, tile_l)
